# speedup vs baseline: 30.5898x; 30.5898x over previous
import functools

import jax
import jax.numpy as jnp
from jax.experimental import pallas as pl
from jax.experimental.pallas import tpu as pltpu

ALPHA = 0.1
NODE_SPLIT = 2
EDGE_TILE = 512
NODE_TILE = 1024


def _node_linear_kernel(x_ref, w_ref, b_ref, o_ref):
    o_ref[...] = (jnp.dot(x_ref[...], w_ref[...],
                          preferred_element_type=jnp.float32)
                  + b_ref[...]).astype(jnp.bfloat16)


def _node_linear(x, w, b):
    n, fi = x.shape
    fo = w.shape[1]
    tn = NODE_TILE
    return pl.pallas_call(
        _node_linear_kernel,
        out_shape=jax.ShapeDtypeStruct((n, fo), jnp.bfloat16),
        grid=(n // tn,),
        in_specs=[pl.BlockSpec((tn, fi), lambda i: (i, 0)),
                  pl.BlockSpec((fi, fo), lambda i: (0, 0)),
                  pl.BlockSpec((1, fo), lambda i: (0, 0))],
        out_specs=pl.BlockSpec((tn, fo), lambda i: (i, 0)),
        compiler_params=pltpu.CompilerParams(
            dimension_semantics=("parallel",)),
    )(x, w, b.reshape(1, fo))


def _edge_agg_kernel(dst_ref, hs_ref, ef_ref, we_ref, be_ref, x_ref,
                     o_ref, acc_ref, *, rows, te, alpha):
    j = pl.program_id(1)

    @pl.when(j == 0)
    def _():
        acc_ref[...] = jnp.zeros_like(acc_ref)

    f = (jnp.dot(ef_ref[...].astype(jnp.bfloat16), we_ref[...],
                 preferred_element_type=jnp.float32) + be_ref[...])
    msg = (hs_ref[...].astype(jnp.float32) * f).astype(jnp.bfloat16)

    i = pl.program_id(0)
    row_ids = i * rows + jax.lax.broadcasted_iota(jnp.int32, (rows, te), 0)
    onehot = (row_ids == dst_ref[0]).astype(jnp.bfloat16)
    acc_ref[...] += jnp.dot(onehot, msg, preferred_element_type=jnp.float32)

    @pl.when(j == pl.num_programs(1) - 1)
    def _():
        o_ref[...] = x_ref[...] + alpha * jnp.maximum(acc_ref[...], 0.0)


def _edge_aggregate(dst, hs, ef, we_bf, be, x, alpha):
    n, fi = x.shape
    e, fe = ef.shape
    o = we_bf.shape[1]
    rows = n // NODE_SPLIT
    te = EDGE_TILE
    c = e // te
    dst3 = dst.reshape(c, 1, te)
    body = functools.partial(_edge_agg_kernel, rows=rows, te=te, alpha=alpha)
    return pl.pallas_call(
        body,
        out_shape=jax.ShapeDtypeStruct((n, o), jnp.float32),
        grid=(NODE_SPLIT, c),
        in_specs=[
            pl.BlockSpec((1, 1, te), lambda i, j: (j, 0, 0)),
            pl.BlockSpec((te, o), lambda i, j: (j, 0)),
            pl.BlockSpec((te, fe), lambda i, j: (j, 0)),
            pl.BlockSpec((fe, o), lambda i, j: (0, 0)),
            pl.BlockSpec((1, o), lambda i, j: (0, 0)),
            pl.BlockSpec((rows, fi), lambda i, j: (i, 0)),
        ],
        out_specs=pl.BlockSpec((rows, o), lambda i, j: (i, 0)),
        scratch_shapes=[pltpu.VMEM((rows, o), jnp.float32)],
        compiler_params=pltpu.CompilerParams(
            dimension_semantics=("parallel", "arbitrary")),
    )(dst3, hs, ef, we_bf, be.reshape(1, o), x)


def kernel(w_node, b_node, w_edge, b_edge, node_feats, edge_feats, src, dst):
    nl = _node_linear(node_feats, w_node, b_node)
    hs = jnp.take(nl, src.astype(jnp.int32), axis=0)
    return _edge_aggregate(dst.astype(jnp.int32), hs, edge_feats,
                           w_edge.astype(jnp.bfloat16), b_edge,
                           node_feats, ALPHA)


# speedup vs baseline: 39.8205x; 1.3018x over previous
import functools

import jax
import jax.numpy as jnp
from jax.experimental import pallas as pl
from jax.experimental.pallas import tpu as pltpu

ALPHA = 0.1
NODE_SPLIT = 2
EDGE_TILE = 512
NODE_TILE = 1024


def _node_linear_kernel(x_ref, w_ref, b_ref, o_ref):
    o_ref[...] = (jnp.dot(x_ref[...], w_ref[...],
                          preferred_element_type=jnp.float32)
                  + b_ref[...])


def _node_linear(x, w, b):
    n, fi = x.shape
    fo = w.shape[1]
    tn = NODE_TILE
    return pl.pallas_call(
        _node_linear_kernel,
        out_shape=jax.ShapeDtypeStruct((n, fo), jnp.float32),
        grid=(n // tn,),
        in_specs=[pl.BlockSpec((tn, fi), lambda i: (i, 0)),
                  pl.BlockSpec((fi, fo), lambda i: (0, 0)),
                  pl.BlockSpec((1, fo), lambda i: (0, 0))],
        out_specs=pl.BlockSpec((tn, fo), lambda i: (i, 0)),
        compiler_params=pltpu.CompilerParams(
            dimension_semantics=("parallel",)),
    )(x, w, b.reshape(1, fo))


def _edge_agg_kernel(src_ref, dst_ref, nl_ref, ef_ref, we_ref, be_ref, x_ref,
                     o_ref, acc_ref, hs_ref, *, rows, te, alpha):
    j = pl.program_id(1)

    @pl.when(j == 0)
    def _():
        acc_ref[...] = jnp.zeros_like(acc_ref)

    f = (jnp.dot(ef_ref[...].astype(jnp.bfloat16), we_ref[...],
                 preferred_element_type=jnp.float32) + be_ref[...])

    for mi in range(te):
        hs_ref[pl.ds(mi, 1), :] = nl_ref[src_ref[0, 0, mi]]

    msg = (hs_ref[...] * f).astype(jnp.bfloat16)

    i = pl.program_id(0)
    row_ids = i * rows + jax.lax.broadcasted_iota(jnp.int32, (rows, te), 0)
    onehot = (row_ids == dst_ref[0]).astype(jnp.bfloat16)
    acc_ref[...] += jnp.dot(onehot, msg, preferred_element_type=jnp.float32)

    @pl.when(j == pl.num_programs(1) - 1)
    def _():
        o_ref[...] = x_ref[...] + alpha * jnp.maximum(acc_ref[...], 0.0)


def _edge_aggregate(src, dst, nl, ef, we_bf, be, x, alpha):
    n, fi = x.shape
    e, fe = ef.shape
    o = we_bf.shape[1]
    rows = n // NODE_SPLIT
    te = EDGE_TILE
    c = e // te
    src3 = src.reshape(c, 1, te)
    dst3 = dst.reshape(c, 1, te)
    nl3 = nl.reshape(n, 1, o)
    body = functools.partial(_edge_agg_kernel, rows=rows, te=te, alpha=alpha)
    return pl.pallas_call(
        body,
        out_shape=jax.ShapeDtypeStruct((n, o), jnp.float32),
        grid=(NODE_SPLIT, c),
        in_specs=[
            pl.BlockSpec((1, 1, te), lambda i, j: (j, 0, 0),
                         memory_space=pltpu.SMEM),
            pl.BlockSpec((1, 1, te), lambda i, j: (j, 0, 0)),
            pl.BlockSpec((n, 1, o), lambda i, j: (0, 0, 0)),
            pl.BlockSpec((te, fe), lambda i, j: (j, 0)),
            pl.BlockSpec((fe, o), lambda i, j: (0, 0)),
            pl.BlockSpec((1, o), lambda i, j: (0, 0)),
            pl.BlockSpec((rows, fi), lambda i, j: (i, 0)),
        ],
        out_specs=pl.BlockSpec((rows, o), lambda i, j: (i, 0)),
        scratch_shapes=[pltpu.VMEM((rows, o), jnp.float32),
                        pltpu.VMEM((te, o), jnp.float32)],
        compiler_params=pltpu.CompilerParams(
            dimension_semantics=("parallel", "arbitrary")),
    )(src3, dst3, nl3, ef, we_bf, be.reshape(1, o), x)


def kernel(w_node, b_node, w_edge, b_edge, node_feats, edge_feats, src, dst):
    nl = _node_linear(node_feats, w_node, b_node)
    return _edge_aggregate(src.astype(jnp.int32), dst.astype(jnp.int32), nl,
                           edge_feats, w_edge.astype(jnp.bfloat16), b_edge,
                           node_feats, ALPHA)


# speedup vs baseline: 49.7588x; 1.2496x over previous
import functools

import jax
import jax.numpy as jnp
from jax.experimental import pallas as pl
from jax.experimental.pallas import tpu as pltpu

ALPHA = 0.1
EDGE_TILE = 512
NODE_TILE = 1024


def _node_linear_kernel(x_ref, w_ref, b_ref, o_ref):
    nl = (jnp.dot(x_ref[...], w_ref[...],
                  preferred_element_type=jnp.float32) + b_ref[...])
    o_ref[...] = nl.reshape(o_ref.shape)


def _node_linear(x, w, b):
    n, fi = x.shape
    fo = w.shape[1]
    tn = NODE_TILE
    return pl.pallas_call(
        _node_linear_kernel,
        out_shape=jax.ShapeDtypeStruct((n, 1, fo), jnp.float32),
        grid=(n // tn,),
        in_specs=[pl.BlockSpec((tn, fi), lambda i: (i, 0)),
                  pl.BlockSpec((fi, fo), lambda i: (0, 0)),
                  pl.BlockSpec((1, fo), lambda i: (0, 0))],
        out_specs=pl.BlockSpec((tn, 1, fo), lambda i: (i, 0, 0)),
        compiler_params=pltpu.CompilerParams(
            dimension_semantics=("parallel",)),
    )(x, w, b.reshape(1, fo))


def _edge_agg_kernel(src_ref, dst_ref, nl_ref, ef_ref, we_ref, be_ref, x_ref,
                     o_ref, acc_ref, hs_ref, *, rows, te, alpha):
    j = pl.program_id(0)

    @pl.when(j == 0)
    def _():
        acc_ref[...] = jnp.zeros_like(acc_ref)

    f = (jnp.dot(ef_ref[...].astype(jnp.bfloat16), we_ref[...],
                 preferred_element_type=jnp.float32) + be_ref[...])

    for mi in range(te):
        hs_ref[pl.ds(mi, 1), :] = nl_ref[src_ref[0, 0, mi]]

    msg = (hs_ref[...] * f).astype(jnp.bfloat16)

    row_ids = jax.lax.broadcasted_iota(jnp.int32, (rows, te), 0)
    onehot = (row_ids == dst_ref[0]).astype(jnp.bfloat16)
    acc_ref[...] += jnp.dot(onehot, msg, preferred_element_type=jnp.float32)

    @pl.when(j == pl.num_programs(0) - 1)
    def _():
        o_ref[...] = x_ref[...] + alpha * jnp.maximum(acc_ref[...], 0.0)


def _edge_aggregate(src, dst, nl3, ef, we_bf, be, x, alpha):
    n, fi = x.shape
    e, fe = ef.shape
    o = we_bf.shape[1]
    rows = n
    te = EDGE_TILE
    c = e // te
    src3 = src.reshape(c, 1, te)
    dst3 = dst.reshape(c, 1, te)
    body = functools.partial(_edge_agg_kernel, rows=rows, te=te, alpha=alpha)
    return pl.pallas_call(
        body,
        out_shape=jax.ShapeDtypeStruct((n, o), jnp.float32),
        grid=(c,),
        in_specs=[
            pl.BlockSpec((1, 1, te), lambda j: (j, 0, 0),
                         memory_space=pltpu.SMEM),
            pl.BlockSpec((1, 1, te), lambda j: (j, 0, 0)),
            pl.BlockSpec((n, 1, o), lambda j: (0, 0, 0)),
            pl.BlockSpec((te, fe), lambda j: (j, 0)),
            pl.BlockSpec((fe, o), lambda j: (0, 0)),
            pl.BlockSpec((1, o), lambda j: (0, 0)),
            pl.BlockSpec((rows, fi), lambda j: (0, 0)),
        ],
        out_specs=pl.BlockSpec((rows, o), lambda j: (0, 0)),
        scratch_shapes=[pltpu.VMEM((rows, o), jnp.float32),
                        pltpu.VMEM((te, o), jnp.float32)],
        compiler_params=pltpu.CompilerParams(
            dimension_semantics=("arbitrary",)),
    )(src3, dst3, nl3, ef, we_bf, be.reshape(1, o), x)


def kernel(w_node, b_node, w_edge, b_edge, node_feats, edge_feats, src, dst):
    nl3 = _node_linear(node_feats, w_node, b_node)
    return _edge_aggregate(src.astype(jnp.int32), dst.astype(jnp.int32), nl3,
                           edge_feats, w_edge.astype(jnp.bfloat16), b_edge,
                           node_feats, ALPHA)


# speedup vs baseline: 65.7716x; 1.3218x over previous
import functools

import jax
import jax.numpy as jnp
from jax.experimental import pallas as pl
from jax.experimental.pallas import tpu as pltpu

ALPHA = 0.1
EDGE_TILE = 512
NODE_TILE = 1024


def _node_linear_kernel(x_ref, w_ref, b_ref, o_ref):
    nl = (jnp.dot(x_ref[...], w_ref[...],
                  preferred_element_type=jnp.float32) + b_ref[...])
    o_ref[...] = nl.reshape(o_ref.shape)


def _node_linear(x, w, b):
    n, fi = x.shape
    fo = w.shape[1]
    tn = NODE_TILE
    return pl.pallas_call(
        _node_linear_kernel,
        out_shape=jax.ShapeDtypeStruct((n, 1, fo), jnp.float32),
        grid=(n // tn,),
        in_specs=[pl.BlockSpec((tn, fi), lambda i: (i, 0)),
                  pl.BlockSpec((fi, fo), lambda i: (0, 0)),
                  pl.BlockSpec((1, fo), lambda i: (0, 0))],
        out_specs=pl.BlockSpec((tn, 1, fo), lambda i: (i, 0, 0)),
        compiler_params=pltpu.CompilerParams(
            dimension_semantics=("parallel",)),
    )(x, w, b.reshape(1, fo))


def _edge_agg_kernel(src_ref, dst_ref, nl_ref, ef_ref, we_ref, be_ref, x_ref,
                     o_ref, acc_ref, hs_ref, *, rows, te, alpha):
    j = pl.program_id(0)

    @pl.when(j == 0)
    def _():
        acc_ref[...] = jnp.zeros_like(acc_ref)

    f = (jnp.dot(ef_ref[...].astype(jnp.bfloat16), we_ref[...],
                 preferred_element_type=jnp.float32) + be_ref[...])

    for mi in range(te):
        hs_ref[pl.ds(mi, 1), :] = nl_ref[src_ref[0, 0, mi]]

    msg_t = jnp.transpose(hs_ref[...] * f).astype(jnp.bfloat16)

    col_ids = jax.lax.broadcasted_iota(jnp.int32, (1, rows), 1)
    onehot_t = (dst_ref[0] == col_ids).astype(jnp.bfloat16)
    acc_ref[...] += jnp.dot(msg_t, onehot_t,
                            preferred_element_type=jnp.float32)

    @pl.when(j == pl.num_programs(0) - 1)
    def _():
        o_ref[...] = x_ref[...] + alpha * jnp.maximum(
            jnp.transpose(acc_ref[...]), 0.0)


def _edge_aggregate(src, dst, nl3, ef, we_bf, be, x, alpha):
    n, fi = x.shape
    e, fe = ef.shape
    o = we_bf.shape[1]
    rows = n
    te = EDGE_TILE
    c = e // te
    src3 = src.reshape(c, 1, te)
    dst3 = dst.reshape(c, te, 1)
    body = functools.partial(_edge_agg_kernel, rows=rows, te=te, alpha=alpha)
    return pl.pallas_call(
        body,
        out_shape=jax.ShapeDtypeStruct((n, o), jnp.float32),
        grid=(c,),
        in_specs=[
            pl.BlockSpec((1, 1, te), lambda j: (j, 0, 0),
                         memory_space=pltpu.SMEM),
            pl.BlockSpec((1, te, 1), lambda j: (j, 0, 0)),
            pl.BlockSpec((n, 1, o), lambda j: (0, 0, 0)),
            pl.BlockSpec((te, fe), lambda j: (j, 0)),
            pl.BlockSpec((fe, o), lambda j: (0, 0)),
            pl.BlockSpec((1, o), lambda j: (0, 0)),
            pl.BlockSpec((rows, fi), lambda j: (0, 0)),
        ],
        out_specs=pl.BlockSpec((rows, o), lambda j: (0, 0)),
        scratch_shapes=[pltpu.VMEM((o, rows), jnp.float32),
                        pltpu.VMEM((te, o), jnp.float32)],
        compiler_params=pltpu.CompilerParams(
            dimension_semantics=("arbitrary",)),
    )(src3, dst3, nl3, ef, we_bf, be.reshape(1, o), x)


def kernel(w_node, b_node, w_edge, b_edge, node_feats, edge_feats, src, dst):
    nl3 = _node_linear(node_feats, w_node, b_node)
    return _edge_aggregate(src.astype(jnp.int32), dst.astype(jnp.int32), nl3,
                           edge_feats, w_edge.astype(jnp.bfloat16), b_edge,
                           node_feats, ALPHA)


# speedup vs baseline: 70.6664x; 1.0744x over previous
import functools

import jax
import jax.numpy as jnp
from jax.experimental import pallas as pl
from jax.experimental.pallas import tpu as pltpu

ALPHA = 0.1
EDGE_TILE = 1024
NODE_TILE = 1024


def _node_linear_kernel(x_ref, w_ref, b_ref, o_ref):
    nl = (jnp.dot(x_ref[...], w_ref[...],
                  preferred_element_type=jnp.float32) + b_ref[...])
    o_ref[...] = nl.reshape(o_ref.shape)


def _node_linear(x, w, b):
    n, fi = x.shape
    fo = w.shape[1]
    tn = NODE_TILE
    return pl.pallas_call(
        _node_linear_kernel,
        out_shape=jax.ShapeDtypeStruct((n, 1, fo), jnp.float32),
        grid=(n // tn,),
        in_specs=[pl.BlockSpec((tn, fi), lambda i: (i, 0)),
                  pl.BlockSpec((fi, fo), lambda i: (0, 0)),
                  pl.BlockSpec((1, fo), lambda i: (0, 0))],
        out_specs=pl.BlockSpec((tn, 1, fo), lambda i: (i, 0, 0)),
        compiler_params=pltpu.CompilerParams(
            dimension_semantics=("parallel",)),
    )(x, w, b.reshape(1, fo))


def _edge_agg_kernel(src_ref, dst_ref, nl_ref, ef_ref, we_ref, be_ref, x_ref,
                     o_ref, acc_ref, hs_ref, *, rows, te, alpha):
    j = pl.program_id(0)

    @pl.when(j == 0)
    def _():
        acc_ref[...] = jnp.zeros_like(acc_ref)

    f = (jnp.dot(ef_ref[...].astype(jnp.bfloat16), we_ref[...],
                 preferred_element_type=jnp.float32) + be_ref[...])

    for mi in range(te):
        hs_ref[pl.ds(mi, 1), :] = nl_ref[src_ref[0, 0, mi]]

    msg_t = jnp.transpose(hs_ref[...] * f).astype(jnp.bfloat16)

    col_ids = jax.lax.broadcasted_iota(jnp.int32, (1, rows), 1)
    onehot_t = (dst_ref[0] == col_ids).astype(jnp.bfloat16)
    acc_ref[...] += jnp.dot(msg_t, onehot_t,
                            preferred_element_type=jnp.float32)

    @pl.when(j == pl.num_programs(0) - 1)
    def _():
        o_ref[...] = x_ref[...] + alpha * jnp.maximum(
            jnp.transpose(acc_ref[...]), 0.0)


def _edge_aggregate(src, dst, nl3, ef, we_bf, be, x, alpha):
    n, fi = x.shape
    e, fe = ef.shape
    o = we_bf.shape[1]
    rows = n
    te = EDGE_TILE
    c = e // te
    src3 = src.reshape(c, 1, te)
    dst3 = dst.reshape(c, te, 1)
    body = functools.partial(_edge_agg_kernel, rows=rows, te=te, alpha=alpha)
    return pl.pallas_call(
        body,
        out_shape=jax.ShapeDtypeStruct((n, o), jnp.float32),
        grid=(c,),
        in_specs=[
            pl.BlockSpec((1, 1, te), lambda j: (j, 0, 0),
                         memory_space=pltpu.SMEM),
            pl.BlockSpec((1, te, 1), lambda j: (j, 0, 0)),
            pl.BlockSpec((n, 1, o), lambda j: (0, 0, 0)),
            pl.BlockSpec((te, fe), lambda j: (j, 0)),
            pl.BlockSpec((fe, o), lambda j: (0, 0)),
            pl.BlockSpec((1, o), lambda j: (0, 0)),
            pl.BlockSpec((rows, fi), lambda j: (0, 0)),
        ],
        out_specs=pl.BlockSpec((rows, o), lambda j: (0, 0)),
        scratch_shapes=[pltpu.VMEM((o, rows), jnp.float32),
                        pltpu.VMEM((te, o), jnp.float32)],
        compiler_params=pltpu.CompilerParams(
            dimension_semantics=("arbitrary",)),
    )(src3, dst3, nl3, ef, we_bf, be.reshape(1, o), x)


def kernel(w_node, b_node, w_edge, b_edge, node_feats, edge_feats, src, dst):
    nl3 = _node_linear(node_feats, w_node, b_node)
    return _edge_aggregate(src.astype(jnp.int32), dst.astype(jnp.int32), nl3,
                           edge_feats, w_edge.astype(jnp.bfloat16), b_edge,
                           node_feats, ALPHA)


# speedup vs baseline: 72.2687x; 1.0227x over previous
import functools

import jax
import jax.numpy as jnp
from jax.experimental import pallas as pl
from jax.experimental.pallas import tpu as pltpu

ALPHA = 0.1
EDGE_TILE = 2048
NODE_TILE = 1024


def _node_linear_kernel(x_ref, w_ref, b_ref, o_ref):
    nl = (jnp.dot(x_ref[...], w_ref[...],
                  preferred_element_type=jnp.float32) + b_ref[...])
    o_ref[...] = nl.reshape(o_ref.shape)


def _node_linear(x, w, b):
    n, fi = x.shape
    fo = w.shape[1]
    tn = NODE_TILE
    return pl.pallas_call(
        _node_linear_kernel,
        out_shape=jax.ShapeDtypeStruct((n, 1, fo), jnp.float32),
        grid=(n // tn,),
        in_specs=[pl.BlockSpec((tn, fi), lambda i: (i, 0)),
                  pl.BlockSpec((fi, fo), lambda i: (0, 0)),
                  pl.BlockSpec((1, fo), lambda i: (0, 0))],
        out_specs=pl.BlockSpec((tn, 1, fo), lambda i: (i, 0, 0)),
        compiler_params=pltpu.CompilerParams(
            dimension_semantics=("parallel",)),
    )(x, w, b.reshape(1, fo))


def _edge_agg_kernel(src_ref, dst_ref, nl_ref, ef_ref, we_ref, be_ref, x_ref,
                     o_ref, acc_ref, hs_ref, *, rows, te, alpha):
    j = pl.program_id(0)

    @pl.when(j == 0)
    def _():
        acc_ref[...] = jnp.zeros_like(acc_ref)

    f = (jnp.dot(ef_ref[...].astype(jnp.bfloat16), we_ref[...],
                 preferred_element_type=jnp.float32) + be_ref[...])

    for mi in range(te):
        hs_ref[pl.ds(mi, 1), :] = nl_ref[src_ref[0, 0, mi]]

    msg_t = jnp.transpose(hs_ref[...] * f).astype(jnp.bfloat16)

    col_ids = jax.lax.broadcasted_iota(jnp.int32, (1, rows), 1)
    onehot_t = (dst_ref[0] == col_ids).astype(jnp.bfloat16)
    acc_ref[...] += jnp.dot(msg_t, onehot_t,
                            preferred_element_type=jnp.float32)

    @pl.when(j == pl.num_programs(0) - 1)
    def _():
        o_ref[...] = x_ref[...] + alpha * jnp.maximum(
            jnp.transpose(acc_ref[...]), 0.0)


def _edge_aggregate(src, dst, nl3, ef, we_bf, be, x, alpha):
    n, fi = x.shape
    e, fe = ef.shape
    o = we_bf.shape[1]
    rows = n
    te = EDGE_TILE
    c = e // te
    src3 = src.reshape(c, 1, te)
    dst3 = dst.reshape(c, te, 1)
    body = functools.partial(_edge_agg_kernel, rows=rows, te=te, alpha=alpha)
    return pl.pallas_call(
        body,
        out_shape=jax.ShapeDtypeStruct((n, o), jnp.float32),
        grid=(c,),
        in_specs=[
            pl.BlockSpec((1, 1, te), lambda j: (j, 0, 0),
                         memory_space=pltpu.SMEM),
            pl.BlockSpec((1, te, 1), lambda j: (j, 0, 0)),
            pl.BlockSpec((n, 1, o), lambda j: (0, 0, 0)),
            pl.BlockSpec((te, fe), lambda j: (j, 0)),
            pl.BlockSpec((fe, o), lambda j: (0, 0)),
            pl.BlockSpec((1, o), lambda j: (0, 0)),
            pl.BlockSpec((rows, fi), lambda j: (0, 0)),
        ],
        out_specs=pl.BlockSpec((rows, o), lambda j: (0, 0)),
        scratch_shapes=[pltpu.VMEM((o, rows), jnp.float32),
                        pltpu.VMEM((te, o), jnp.float32)],
        compiler_params=pltpu.CompilerParams(
            dimension_semantics=("arbitrary",)),
    )(src3, dst3, nl3, ef, we_bf, be.reshape(1, o), x)


def kernel(w_node, b_node, w_edge, b_edge, node_feats, edge_feats, src, dst):
    nl3 = _node_linear(node_feats, w_node, b_node)
    return _edge_aggregate(src.astype(jnp.int32), dst.astype(jnp.int32), nl3,
                           edge_feats, w_edge.astype(jnp.bfloat16), b_edge,
                           node_feats, ALPHA)


# speedup vs baseline: 79.0164x; 1.0934x over previous
import functools

import jax
import jax.numpy as jnp
from jax.experimental import pallas as pl
from jax.experimental.pallas import tpu as pltpu

ALPHA = 0.1
EDGE_TILE = 2048
NODE_TILE = 1024


def _node_linear_kernel(x_ref, w_ref, b_ref, o_ref):
    nl = (jnp.dot(x_ref[...], w_ref[...],
                  preferred_element_type=jnp.float32) + b_ref[...])
    o_ref[...] = nl.reshape(o_ref.shape)


def _node_linear(x, w, b):
    n, fi = x.shape
    fo = w.shape[1]
    tn = NODE_TILE
    return pl.pallas_call(
        _node_linear_kernel,
        out_shape=jax.ShapeDtypeStruct((n, 1, fo), jnp.float32),
        grid=(n // tn,),
        in_specs=[pl.BlockSpec((tn, fi), lambda i: (i, 0)),
                  pl.BlockSpec((fi, fo), lambda i: (0, 0)),
                  pl.BlockSpec((1, fo), lambda i: (0, 0))],
        out_specs=pl.BlockSpec((tn, 1, fo), lambda i: (i, 0, 0)),
        compiler_params=pltpu.CompilerParams(
            dimension_semantics=("parallel",)),
    )(x, w, b.reshape(1, fo))


def _edge_agg_kernel(src_ref, dst_ref, nl_ref, ef_ref, we_ref, be_ref, x_ref,
                     o_ref, acc_ref, hs_ref, *, rows, te, alpha):
    j = pl.program_id(0)

    @pl.when(j == 0)
    def _():
        acc_ref[...] = jnp.zeros_like(acc_ref)

    f = (jnp.dot(ef_ref[...].astype(jnp.bfloat16), we_ref[...],
                 preferred_element_type=jnp.float32) + be_ref[...])

    for mi in range(te):
        hs_ref[pl.ds(mi, 1), :] = nl_ref[src_ref[0, 0, mi]]

    msg_t = jnp.transpose(hs_ref[...] * f).astype(jnp.bfloat16)

    dst_col = jnp.transpose(dst_ref[0])
    col_ids = jax.lax.broadcasted_iota(jnp.int32, (1, rows), 1)
    onehot_t = (dst_col == col_ids).astype(jnp.bfloat16)
    acc_ref[...] += jnp.dot(msg_t, onehot_t,
                            preferred_element_type=jnp.float32)

    @pl.when(j == pl.num_programs(0) - 1)
    def _():
        o_ref[...] = x_ref[...] + alpha * jnp.maximum(
            jnp.transpose(acc_ref[...]), 0.0)


def _edge_aggregate(src, dst, nl3, ef, we_bf, be, x, alpha):
    n, fi = x.shape
    e, fe = ef.shape
    o = we_bf.shape[1]
    rows = n
    te = EDGE_TILE
    c = e // te
    src3 = src.reshape(c, 1, te)
    dst3 = dst.reshape(c, 1, te)
    body = functools.partial(_edge_agg_kernel, rows=rows, te=te, alpha=alpha)
    return pl.pallas_call(
        body,
        out_shape=jax.ShapeDtypeStruct((n, o), jnp.float32),
        grid=(c,),
        in_specs=[
            pl.BlockSpec((1, 1, te), lambda j: (j, 0, 0),
                         memory_space=pltpu.SMEM),
            pl.BlockSpec((1, 1, te), lambda j: (j, 0, 0)),
            pl.BlockSpec((n, 1, o), lambda j: (0, 0, 0)),
            pl.BlockSpec((te, fe), lambda j: (j, 0)),
            pl.BlockSpec((fe, o), lambda j: (0, 0)),
            pl.BlockSpec((1, o), lambda j: (0, 0)),
            pl.BlockSpec((rows, fi), lambda j: (0, 0)),
        ],
        out_specs=pl.BlockSpec((rows, o), lambda j: (0, 0)),
        scratch_shapes=[pltpu.VMEM((o, rows), jnp.float32),
                        pltpu.VMEM((te, o), jnp.float32)],
        compiler_params=pltpu.CompilerParams(
            dimension_semantics=("arbitrary",)),
    )(src3, dst3, nl3, ef, we_bf, be.reshape(1, o), x)


def kernel(w_node, b_node, w_edge, b_edge, node_feats, edge_feats, src, dst):
    nl3 = _node_linear(node_feats, w_node, b_node)
    return _edge_aggregate(src.astype(jnp.int32), dst.astype(jnp.int32), nl3,
                           edge_feats, w_edge.astype(jnp.bfloat16), b_edge,
                           node_feats, ALPHA)


# speedup vs baseline: 80.5017x; 1.0188x over previous
import functools

import jax
import jax.numpy as jnp
from jax.experimental import pallas as pl
from jax.experimental.pallas import tpu as pltpu

ALPHA = 0.1
EDGE_TILE = 2048
NODE_TILE = 1024


def _node_linear_kernel(x_ref, w_ref, b_ref, o_ref):
    nl = (jnp.dot(x_ref[...], w_ref[...],
                  preferred_element_type=jnp.float32) + b_ref[...])
    o_ref[...] = nl.reshape(o_ref.shape)


def _node_linear(x, w, b):
    n, fi = x.shape
    fo = w.shape[1]
    tn = NODE_TILE
    return pl.pallas_call(
        _node_linear_kernel,
        out_shape=jax.ShapeDtypeStruct((n, 1, fo), jnp.float32),
        grid=(n // tn,),
        in_specs=[pl.BlockSpec((tn, fi), lambda i: (i, 0)),
                  pl.BlockSpec((fi, fo), lambda i: (0, 0)),
                  pl.BlockSpec((1, fo), lambda i: (0, 0))],
        out_specs=pl.BlockSpec((tn, 1, fo), lambda i: (i, 0, 0)),
        compiler_params=pltpu.CompilerParams(
            dimension_semantics=("parallel",)),
    )(x, w, b.reshape(1, fo))


def _edge_agg_kernel(src_ref, dst_ref, wn_ref, bn_ref, ef_ref, we_ref, be_ref,
                     x_ref, o_ref, acc_ref, hs_ref, nl_ref, *, rows, te, alpha):
    j = pl.program_id(0)

    @pl.when(j == 0)
    def _():
        acc_ref[...] = jnp.zeros_like(acc_ref)
        nl = (jnp.dot(x_ref[...], wn_ref[...],
                      preferred_element_type=jnp.float32) + bn_ref[...])
        nl_ref[...] = nl.reshape(nl_ref.shape)

    f = (jnp.dot(ef_ref[...].astype(jnp.bfloat16), we_ref[...],
                 preferred_element_type=jnp.float32) + be_ref[...])

    for mi in range(te):
        hs_ref[pl.ds(mi, 1), :] = nl_ref[src_ref[0, 0, mi]]

    msg_t = jnp.transpose(hs_ref[...] * f).astype(jnp.bfloat16)

    dst_col = jnp.transpose(dst_ref[0])
    col_ids = jax.lax.broadcasted_iota(jnp.int32, (1, rows), 1)
    onehot_t = (dst_col == col_ids).astype(jnp.bfloat16)
    acc_ref[...] += jnp.dot(msg_t, onehot_t,
                            preferred_element_type=jnp.float32)

    @pl.when(j == pl.num_programs(0) - 1)
    def _():
        o_ref[...] = x_ref[...] + alpha * jnp.maximum(
            jnp.transpose(acc_ref[...]), 0.0)


def _edge_aggregate(src, dst, wn, bn, ef, we_bf, be, x, alpha):
    n, fi = x.shape
    e, fe = ef.shape
    o = we_bf.shape[1]
    rows = n
    te = EDGE_TILE
    c = e // te
    src3 = src.reshape(c, 1, te)
    dst3 = dst.reshape(c, 1, te)
    body = functools.partial(_edge_agg_kernel, rows=rows, te=te, alpha=alpha)
    return pl.pallas_call(
        body,
        out_shape=jax.ShapeDtypeStruct((n, o), jnp.float32),
        grid=(c,),
        in_specs=[
            pl.BlockSpec((1, 1, te), lambda j: (j, 0, 0),
                         memory_space=pltpu.SMEM),
            pl.BlockSpec((1, 1, te), lambda j: (j, 0, 0)),
            pl.BlockSpec((fi, o), lambda j: (0, 0)),
            pl.BlockSpec((1, o), lambda j: (0, 0)),
            pl.BlockSpec((te, fe), lambda j: (j, 0)),
            pl.BlockSpec((fe, o), lambda j: (0, 0)),
            pl.BlockSpec((1, o), lambda j: (0, 0)),
            pl.BlockSpec((rows, fi), lambda j: (0, 0)),
        ],
        out_specs=pl.BlockSpec((rows, o), lambda j: (0, 0)),
        scratch_shapes=[pltpu.VMEM((o, rows), jnp.float32),
                        pltpu.VMEM((te, o), jnp.float32),
                        pltpu.VMEM((n, 1, o), jnp.float32)],
        compiler_params=pltpu.CompilerParams(
            dimension_semantics=("arbitrary",)),
    )(src3, dst3, wn, bn.reshape(1, o), ef, we_bf, be.reshape(1, o), x)


def kernel(w_node, b_node, w_edge, b_edge, node_feats, edge_feats, src, dst):
    return _edge_aggregate(src.astype(jnp.int32), dst.astype(jnp.int32),
                           w_node, b_node, edge_feats,
                           w_edge.astype(jnp.bfloat16), b_edge,
                           node_feats, ALPHA)


# speedup vs baseline: 88.4346x; 1.0985x over previous
import functools

import jax
import jax.numpy as jnp
from jax.experimental import pallas as pl
from jax.experimental.pallas import tpu as pltpu

ALPHA = 0.1
EDGE_TILE = 2048
NODE_TILE = 1024


def _node_linear_kernel(x_ref, w_ref, b_ref, o_ref):
    nl = (jnp.dot(x_ref[...], w_ref[...],
                  preferred_element_type=jnp.float32) + b_ref[...])
    o_ref[...] = nl.reshape(o_ref.shape)


def _node_linear(x, w, b):
    n, fi = x.shape
    fo = w.shape[1]
    tn = NODE_TILE
    return pl.pallas_call(
        _node_linear_kernel,
        out_shape=jax.ShapeDtypeStruct((n, 1, fo), jnp.float32),
        grid=(n // tn,),
        in_specs=[pl.BlockSpec((tn, fi), lambda i: (i, 0)),
                  pl.BlockSpec((fi, fo), lambda i: (0, 0)),
                  pl.BlockSpec((1, fo), lambda i: (0, 0))],
        out_specs=pl.BlockSpec((tn, 1, fo), lambda i: (i, 0, 0)),
        compiler_params=pltpu.CompilerParams(
            dimension_semantics=("parallel",)),
    )(x, w, b.reshape(1, fo))


def _edge_agg_kernel(src_ref, dst_ref, wn_ref, bn_ref, ef_ref, we_ref, be_ref,
                     x_ref, o_ref, acc_ref, hs_ref, nl_ref, *, rows, te, alpha):
    j = pl.program_id(0)

    @pl.when(j == 0)
    def _():
        acc_ref[...] = jnp.zeros_like(acc_ref)
        nl = (jnp.dot(x_ref[...], wn_ref[...],
                      preferred_element_type=jnp.float32) + bn_ref[...])
        nl_ref[...] = nl.reshape(nl_ref.shape)

    f_t = (jnp.dot(we_ref[...], ef_ref[...].astype(jnp.bfloat16),
                   preferred_element_type=jnp.float32)
           + jnp.transpose(be_ref[...]))

    for mi in range(te):
        hs_ref[pl.ds(mi, 1), :] = nl_ref[src_ref[0, 0, mi]]

    msg_t = (jnp.transpose(hs_ref[...]) * f_t).astype(jnp.bfloat16)

    dst_col = jnp.transpose(dst_ref[0])
    col_ids = jax.lax.broadcasted_iota(jnp.int32, (1, rows), 1)
    onehot_t = (dst_col == col_ids).astype(jnp.bfloat16)
    acc_ref[...] += jnp.dot(msg_t, onehot_t,
                            preferred_element_type=jnp.float32)

    @pl.when(j == pl.num_programs(0) - 1)
    def _():
        o_ref[...] = x_ref[...] + alpha * jnp.maximum(
            jnp.transpose(acc_ref[...]), 0.0)


def _edge_aggregate(src, dst, wn, bn, ef_t, we_t_bf, be, x, alpha):
    n, fi = x.shape
    fe, e = ef_t.shape
    o = we_t_bf.shape[0]
    rows = n
    te = EDGE_TILE
    c = e // te
    src3 = src.reshape(c, 1, te)
    dst3 = dst.reshape(c, 1, te)
    body = functools.partial(_edge_agg_kernel, rows=rows, te=te, alpha=alpha)
    return pl.pallas_call(
        body,
        out_shape=jax.ShapeDtypeStruct((n, o), jnp.float32),
        grid=(c,),
        in_specs=[
            pl.BlockSpec((1, 1, te), lambda j: (j, 0, 0),
                         memory_space=pltpu.SMEM),
            pl.BlockSpec((1, 1, te), lambda j: (j, 0, 0)),
            pl.BlockSpec((fi, o), lambda j: (0, 0)),
            pl.BlockSpec((1, o), lambda j: (0, 0)),
            pl.BlockSpec((fe, te), lambda j: (0, j)),
            pl.BlockSpec((o, fe), lambda j: (0, 0)),
            pl.BlockSpec((1, o), lambda j: (0, 0)),
            pl.BlockSpec((rows, fi), lambda j: (0, 0)),
        ],
        out_specs=pl.BlockSpec((rows, o), lambda j: (0, 0)),
        scratch_shapes=[pltpu.VMEM((o, rows), jnp.float32),
                        pltpu.VMEM((te, o), jnp.float32),
                        pltpu.VMEM((n, 1, o), jnp.float32)],
        compiler_params=pltpu.CompilerParams(
            dimension_semantics=("arbitrary",)),
    )(src3, dst3, wn, bn.reshape(1, o), ef_t, we_t_bf, be.reshape(1, o), x)


def kernel(w_node, b_node, w_edge, b_edge, node_feats, edge_feats, src, dst):
    return _edge_aggregate(src.astype(jnp.int32), dst.astype(jnp.int32),
                           w_node, b_node, edge_feats.T,
                           w_edge.T.astype(jnp.bfloat16), b_edge,
                           node_feats, ALPHA)
